# baseline (speedup 1.0000x reference)
import functools

import jax
import jax.numpy as jnp
from jax import lax
from jax.experimental import pallas as pl
from jax.experimental.pallas import tpu as pltpu
from jax.experimental.pallas import tpu_sc as plsc

NC = 2
NS = 16
EB = 80


def _sc_aggregate(y_flat, src_rows, dst_rows, t_steps, n_pad, d,
                  do_gather=True):
  nslab, slab = src_rows.shape[1], src_rows.shape[2]
  tot = nslab * slab * EB
  BB = 128
  crows = (tot + BB + 16 + BB - 1) // BB
  n_half = n_pad // NC
  acc_rows = n_half + 8
  fl_t = n_half // NS
  zrows = fl_t // 16
  mesh = plsc.VectorSubcoreMesh(core_axis_name="c", subcore_axis_name="s")

  @functools.partial(
      pl.kernel,
      out_type=jax.ShapeDtypeStruct((t_steps, n_pad, d), jnp.float32),
      mesh=mesh,
      compiler_params=pltpu.CompilerParams(needs_layout_passes=False),
      scratch_types=[
          pltpu.VMEM((slab, EB), jnp.int32),
          pltpu.VMEM((slab, EB), jnp.int32),
          pltpu.VMEM((crows, BB), jnp.int32),
          pltpu.VMEM((2, BB), jnp.int32),
          pltpu.VMEM((2, BB), jnp.int32),
          pltpu.VMEM((BB, d), jnp.float32),
          pltpu.VMEM((BB, d), jnp.float32),
          pltpu.VMEM((zrows, d), jnp.float32),
          pltpu.VMEM_SHARED((acc_rows, d), jnp.float32),
          pltpu.SemaphoreType.DMA,
          pltpu.SemaphoreType.DMA,
      ],
  )
  def k(y_hbm, src_hbm, dst_hbm, out_hbm,
        sslab, dslab, cpk, gidx2, sidx2, gbuf0, gbuf1, zero_v, acc_sh,
        sem0, sem1):
    c = lax.axis_index("c")
    s = lax.axis_index("s")
    nv = d // 16
    nch = BB // 16
    sch = slab * EB // 16

    def fill_zero(i, _):
      zero_v[i // nv, pl.ds((i % nv) * 16, 16)] = jnp.zeros((16,), jnp.float32)
      return 0
    lax.fori_loop(0, zrows * nv, fill_zero, 0)

    if not do_gather:
      def fill_one(i, _):
        one = jnp.ones((16,), jnp.float32)
        gbuf0[i // nv, pl.ds((i % nv) * 16, 16)] = one
        gbuf1[i // nv, pl.ds((i % nv) * 16, 16)] = one
        return 0
      lax.fori_loop(0, BB * nv, fill_one, 0)

    base = jnp.broadcast_to(c * n_half, (16,)).astype(jnp.int32)
    zero16 = jnp.zeros((16,), jnp.int32)
    nh16 = jnp.full((16,), n_half, jnp.int32)
    bb16 = jnp.full((16,), BB, jnp.int32)
    pk16 = jnp.full((16,), 32768, jnp.int32)

    def do_slab(sk, off):
      pltpu.sync_copy(src_hbm.at[s, sk], sslab)
      pltpu.sync_copy(dst_hbm.at[s, sk], dslab)

      def compact(ch, off):
        j = ch // (EB // 16)
        sl = pl.ds((ch % (EB // 16)) * 16, 16)
        loc = dslab[j, sl] - base
        ok = (loc >= zero16) & (loc < nh16)
        pk = loc * pk16 + sslab[j, sl]
        cum = plsc.cumsum(jnp.where(ok, 1, 0).astype(jnp.int32))
        pos = cum + jnp.full((16,), off - 1, jnp.int32)
        plsc.store_scatter(cpk, [pos // bb16, pos % bb16], pk, mask=ok)
        return off + jnp.max(cum)
      return lax.fori_loop(0, sch, compact, off)
    kcnt = lax.fori_loop(0, nslab, do_slab, jnp.int32(0))

    trash16 = jnp.full((16,), n_half * 32768, jnp.int32)
    lane = lax.iota(jnp.int32, 16)
    for pc in range(nch):
      pos = jnp.full((16,), kcnt + pc * 16, jnp.int32) + lane
      plsc.store_scatter(cpk, [pos // bb16, pos % bb16], trash16)
    nblk = (kcnt + (BB - 1)) // BB

    r0 = s * fl_t
    for z in range(16):
      pltpu.sync_copy(zero_v, acc_sh.at[pl.ds(r0 + z * zrows, zrows)])
    pltpu.sync_copy(zero_v.at[pl.ds(0, 8)],
                    acc_sh.at[pl.ds(n_half, 8)])

    gbufs = (gbuf0, gbuf1)
    sems = (sem0, sem1)
    m15 = jnp.full((16,), 32767, jnp.int32)

    def unpack(j, b, toff):
      def ch_body(kk, _):
        sl = pl.ds(kk * 16, 16)
        pk = cpk[j, sl]
        gidx2[b, sl] = (pk & m15) + toff
        sidx2[b, sl] = lax.shift_right_logical(pk, jnp.full((16,), 15,
                                                           jnp.int32))
        return 0
      lax.fori_loop(0, nch, ch_body, 0)

    def gather_view(b):
      return y_hbm.at[gidx2.at[b]]

    for t in range(t_steps):
      toff = jnp.full((16,), t * n_pad, jnp.int32)
      plsc.subcore_barrier()

      if do_gather:
        @pl.when(nblk > 0)
        def _():
          unpack(0, 0, toff)
          pltpu.async_copy(gather_view(0), gbuf0, sem0)
      else:
        @pl.when(nblk > 0)
        def _():
          unpack(0, 0, toff)

      def pair(jj, _):
        for b in range(2):
          j = jj * 2 + b

          @pl.when(j < nblk)
          def _():
            if do_gather:
              pltpu.make_async_copy(gather_view(b), gbufs[b], sems[b]).wait()

            @pl.when(j + 1 < nblk)
            def _():
              unpack(j + 1, 1 - b, toff)
              if do_gather:
                pltpu.async_copy(gather_view(1 - b), gbufs[1 - b],
                                 sems[1 - b])
            pltpu.sync_copy(gbufs[b], acc_sh.at[sidx2.at[b]], add=True)
        return 0
      lax.fori_loop(0, (nblk + 1) // 2, pair, 0)
      plsc.subcore_barrier()

      pltpu.sync_copy(acc_sh.at[pl.ds(r0, fl_t)],
                      out_hbm.at[t, pl.ds(c * n_half + r0, fl_t)])

      if t + 1 < t_steps:
        for z in range(16):
          pltpu.sync_copy(zero_v, acc_sh.at[pl.ds(r0 + z * zrows, zrows)])

  return k(y_flat, src_rows, dst_rows)


def _tc_scale(x_pad, degn):
  t_steps, n_pad, d = x_pad.shape
  blk = 2048

  def body(x_ref, degn_ref, y_ref):
    deg = degn_ref[:, 0:1] + 1.0
    dinv = lax.rsqrt(jnp.maximum(deg, 1.0))
    y_ref[...] = x_ref[...] * dinv

  return pl.pallas_call(
      body,
      grid=(t_steps, n_pad // blk),
      in_specs=[
          pl.BlockSpec((1, blk, d), lambda t, i: (t, i, 0)),
          pl.BlockSpec((blk, d), lambda t, i: (i, 0)),
      ],
      out_specs=pl.BlockSpec((1, blk, d), lambda t, i: (t, i, 0)),
      out_shape=jax.ShapeDtypeStruct((t_steps, n_pad, d), jnp.float32),
  )(x_pad, degn)


def _tc_fused(y, p, degn, batch_pad, W_gcn, b_gcn, W_ih, W_hh, b_ih, b_hh,
              fc_W, fc_b, g_segs):
  t_steps, n_pad, d = y.shape
  h_dim = W_gcn.shape[0]
  c_dim = fc_W.shape[0]
  bn = 256
  nblk = n_pad // bn

  def dot_t(a, w):
    return lax.dot_general(a, w, (((1,), (1,)), ((), ())),
                           preferred_element_type=jnp.float32)

  def body(y_ref, p_ref, degn_ref, batch_ref, wg_ref, bg_ref, wih_ref,
           whh_ref, bih_ref, bhh_ref, fcw_ref, fcb_ref, out_ref,
           sums_sc, cnt_sc):
    i = pl.program_id(0)

    @pl.when(i == 0)
    def _():
      sums_sc[...] = jnp.zeros_like(sums_sc)
      cnt_sc[...] = jnp.zeros_like(cnt_sc)

    deg = degn_ref[:, 0:1] + 1.0
    dinv = lax.rsqrt(jnp.maximum(deg, 1.0))

    h = jnp.zeros((bn, h_dim), jnp.float32)
    for t in range(t_steps):
      agg = dinv * (p_ref[t] + y_ref[t])
      zt = jnp.maximum(dot_t(agg, wg_ref[...]) + bg_ref[...], 0.0)
      gi = dot_t(zt, wih_ref[...]) + bih_ref[...]
      gh = dot_t(h, whh_ref[...]) + bhh_ref[...]
      r = jax.nn.sigmoid(gi[:, :h_dim] + gh[:, :h_dim])
      z = jax.nn.sigmoid(gi[:, h_dim:2 * h_dim] + gh[:, h_dim:2 * h_dim])
      nn_ = jnp.tanh(gi[:, 2 * h_dim:] + r * gh[:, 2 * h_dim:])
      h = (1.0 - z) * nn_ + z * h

    out = dot_t(h, fcw_ref[...]) + fcb_ref[...]

    g_iota = lax.broadcasted_iota(jnp.int32, (bn, g_segs), 1)
    oneh = (batch_ref[...] == g_iota).astype(jnp.float32)
    part = lax.dot_general(oneh, out, (((0,), (0,)), ((), ())),
                           preferred_element_type=jnp.float32)
    pcnt = lax.dot_general(oneh, jnp.ones((bn, c_dim), jnp.float32),
                           (((0,), (0,)), ((), ())),
                           preferred_element_type=jnp.float32)
    sums_sc[...] += part
    cnt_sc[...] += pcnt
    out_ref[...] = sums_sc[...] / jnp.maximum(cnt_sc[...], 1.0)

  full = lambda shape: pl.BlockSpec(shape, lambda i: tuple(0 for _ in shape))
  return pl.pallas_call(
      body,
      grid=(nblk,),
      in_specs=[
          pl.BlockSpec((t_steps, bn, d), lambda i: (0, i, 0)),
          pl.BlockSpec((t_steps, bn, d), lambda i: (0, i, 0)),
          pl.BlockSpec((bn, d), lambda i: (i, 0)),
          pl.BlockSpec((bn, 1), lambda i: (i, 0)),
          full((h_dim, d)),
          full((1, h_dim)),
          full((3 * h_dim, h_dim)),
          full((3 * h_dim, h_dim)),
          full((1, 3 * h_dim)),
          full((1, 3 * h_dim)),
          full((c_dim, h_dim)),
          full((1, c_dim)),
      ],
      out_specs=pl.BlockSpec((g_segs, c_dim), lambda i: (0, 0)),
      out_shape=jax.ShapeDtypeStruct((g_segs, c_dim), jnp.float32),
      scratch_shapes=[
          pltpu.VMEM((g_segs, c_dim), jnp.float32),
          pltpu.VMEM((g_segs, c_dim), jnp.float32),
      ],
  )(y, p, degn, batch_pad, W_gcn, b_gcn, W_ih, W_hh, b_ih, b_hh, fc_W, fc_b)


def kernel(x_seq, edge_index, batch, W_gcn, b_gcn, W_ih, W_hh, b_ih, b_hh,
           fc_W, fc_b):
  t_steps, n, d = x_seq.shape
  e = edge_index.shape[1]
  g_segs = 64
  n_pad = 10240

  slab = 10
  nslab = e // (NS * slab * EB)
  src_rows = edge_index[0].reshape(NS, nslab, slab, EB)
  dst_rows = edge_index[1].reshape(NS, nslab, slab, EB)

  dummy = jnp.zeros((8, d), jnp.float32)
  degn = _sc_aggregate(dummy, dst_rows, dst_rows, 1, n_pad, d,
                       do_gather=False)[0]

  x_pad = jnp.pad(x_seq, ((0, 0), (0, n_pad - n), (0, 0)))
  y = _tc_scale(x_pad, degn)

  y_flat = y.reshape(t_steps * n_pad, d)
  p = _sc_aggregate(y_flat, src_rows, dst_rows, t_steps, n_pad, d)

  batch_pad = jnp.pad(batch, (0, n_pad - n),
                      constant_values=g_segs).reshape(n_pad, 1)
  b_gcn2 = b_gcn.reshape(1, -1)
  b_ih2 = b_ih.reshape(1, -1)
  b_hh2 = b_hh.reshape(1, -1)
  fc_b2 = fc_b.reshape(1, -1)

  return _tc_fused(y, p, degn, batch_pad, W_gcn, b_gcn2, W_ih, W_hh,
                   b_ih2, b_hh2, fc_W, fc_b2, g_segs)


# speedup vs baseline: 1.1919x; 1.1919x over previous
import functools

import jax
import jax.numpy as jnp
from jax import lax
from jax.experimental import pallas as pl
from jax.experimental.pallas import tpu as pltpu
from jax.experimental.pallas import tpu_sc as plsc

NC = 2
NS = 16
EB = 80


def _sc_aggregate(y_flat, src_rows, dst_rows, t_steps, n_pad, d,
                  do_gather=True):
  rpt = src_rows.shape[1]
  srows = rpt + 2
  n_half = n_pad // NC
  acc_rows = n_half + 8
  fl_t = n_half // NS
  zrows = fl_t // 16
  mesh = plsc.VectorSubcoreMesh(core_axis_name="c", subcore_axis_name="s")

  @functools.partial(
      pl.kernel,
      out_type=jax.ShapeDtypeStruct((t_steps, n_pad, d), jnp.float32),
      mesh=mesh,
      compiler_params=pltpu.CompilerParams(needs_layout_passes=False),
      scratch_types=[
          pltpu.VMEM((srows, EB), jnp.int32),
          pltpu.VMEM((srows, EB), jnp.int32),
          pltpu.VMEM((EB, d), jnp.float32),
          pltpu.VMEM((EB, d), jnp.float32),
          pltpu.VMEM((zrows, d), jnp.float32),
          pltpu.VMEM_SHARED((acc_rows, d), jnp.float32),
          pltpu.SemaphoreType.DMA,
          pltpu.SemaphoreType.DMA,
      ],
  )
  def k(y_hbm, src_hbm, dst_hbm, out_hbm,
        sv, dv, gbuf0, gbuf1, zero_v, acc_sh, sem0, sem1):
    c = lax.axis_index("c")
    s = lax.axis_index("s")
    nv = d // 16
    nch = EB // 16

    def fill_zero(i, _):
      zero_v[i // nv, pl.ds((i % nv) * 16, 16)] = jnp.zeros((16,), jnp.float32)
      return 0
    lax.fori_loop(0, zrows * nv, fill_zero, 0)

    if not do_gather:
      def fill_one(i, _):
        one = jnp.ones((16,), jnp.float32)
        gbuf0[i // nv, pl.ds((i % nv) * 16, 16)] = one
        gbuf1[i // nv, pl.ds((i % nv) * 16, 16)] = one
        return 0
      lax.fori_loop(0, EB * nv, fill_one, 0)

    pltpu.sync_copy(src_hbm.at[s], sv.at[pl.ds(0, rpt)])
    pltpu.sync_copy(dst_hbm.at[s], dv.at[pl.ds(0, rpt)])

    base = jnp.broadcast_to(c * n_half, (16,)).astype(jnp.int32)
    zero16 = jnp.zeros((16,), jnp.int32)
    nh16 = jnp.full((16,), n_half, jnp.int32)
    eb16 = jnp.full((16,), EB, jnp.int32)

    def compact(ch, off):
      j = ch // nch
      sl = pl.ds((ch % nch) * 16, 16)
      loc = dv[j, sl] - base
      ss = sv[j, sl]
      ok = (loc >= zero16) & (loc < nh16)
      cum = plsc.cumsum(jnp.where(ok, 1, 0).astype(jnp.int32))
      pos = cum + jnp.full((16,), off - 1, jnp.int32)
      prow = pos // eb16
      pcol = pos % eb16
      plsc.store_scatter(dv, [prow, pcol], loc, mask=ok)
      plsc.store_scatter(sv, [prow, pcol], ss, mask=ok)
      return off + jnp.max(cum)
    kcnt = lax.fori_loop(0, rpt * nch, compact, jnp.int32(0))

    trash = jnp.full((16,), n_half, jnp.int32)
    lane = lax.iota(jnp.int32, 16)
    for pc in range(nch):
      pos = jnp.full((16,), kcnt + pc * 16, jnp.int32) + lane
      prow = pos // eb16
      pcol = pos % eb16
      plsc.store_scatter(dv, [prow, pcol], trash)
      plsc.store_scatter(sv, [prow, pcol], zero16)
    nblk = (kcnt + (EB - 1)) // EB

    r0 = s * fl_t
    for z in range(16):
      pltpu.sync_copy(zero_v, acc_sh.at[pl.ds(r0 + z * zrows, zrows)])
    pltpu.sync_copy(zero_v.at[pl.ds(0, 8)],
                    acc_sh.at[pl.ds(n_half, 8)])

    step = jnp.full((16,), n_pad, jnp.int32)
    gbufs = (gbuf0, gbuf1)
    sems = (sem0, sem1)

    def gather_view(j):
      return y_hbm.at[sv.at[j]]

    for t in range(t_steps):
      plsc.subcore_barrier()

      if do_gather:
        @pl.when(nblk > 0)
        def _():
          pltpu.async_copy(gather_view(0), gbuf0, sem0)

      def pair(jj, _):
        for b in range(2):
          j = jj * 2 + b

          @pl.when(j < nblk)
          def _():
            if do_gather:
              @pl.when(j + 1 < nblk)
              def _():
                pltpu.async_copy(gather_view(j + 1), gbufs[1 - b],
                                 sems[1 - b])
              pltpu.make_async_copy(gather_view(j), gbufs[b], sems[b]).wait()
            pltpu.sync_copy(gbufs[b], acc_sh.at[dv.at[j]], add=True)
        return 0
      lax.fori_loop(0, (nblk + 1) // 2, pair, 0)
      plsc.subcore_barrier()

      pltpu.sync_copy(acc_sh.at[pl.ds(r0, fl_t)],
                      out_hbm.at[t, pl.ds(c * n_half + r0, fl_t)])

      if t + 1 < t_steps:
        for z in range(16):
          pltpu.sync_copy(zero_v, acc_sh.at[pl.ds(r0 + z * zrows, zrows)])

        def adv(ch, _):
          sv[ch // nch, pl.ds((ch % nch) * 16, 16)] = (
              sv[ch // nch, pl.ds((ch % nch) * 16, 16)] + step)
          return 0
        lax.fori_loop(0, nblk * nch, adv, 0)

  return k(y_flat, src_rows, dst_rows)


def _tc_scale(x_pad, degn):
  t_steps, n_pad, d = x_pad.shape
  blk = 2048

  def body(x_ref, degn_ref, y_ref):
    deg = degn_ref[:, 0:1] + 1.0
    dinv = lax.rsqrt(jnp.maximum(deg, 1.0))
    y_ref[...] = x_ref[...] * dinv

  return pl.pallas_call(
      body,
      grid=(t_steps, n_pad // blk),
      in_specs=[
          pl.BlockSpec((1, blk, d), lambda t, i: (t, i, 0)),
          pl.BlockSpec((blk, d), lambda t, i: (i, 0)),
      ],
      out_specs=pl.BlockSpec((1, blk, d), lambda t, i: (t, i, 0)),
      out_shape=jax.ShapeDtypeStruct((t_steps, n_pad, d), jnp.float32),
  )(x_pad, degn)


def _tc_fused(y, p, degn, batch_pad, W_gcn, b_gcn, W_ih, W_hh, b_ih, b_hh,
              fc_W, fc_b, g_segs):
  t_steps, n_pad, d = y.shape
  h_dim = W_gcn.shape[0]
  c_dim = fc_W.shape[0]
  bn = 256
  nblk = n_pad // bn

  def dot_t(a, w):
    return lax.dot_general(a, w, (((1,), (1,)), ((), ())),
                           preferred_element_type=jnp.float32)

  def body(y_ref, p_ref, degn_ref, batch_ref, wg_ref, bg_ref, wih_ref,
           whh_ref, bih_ref, bhh_ref, fcw_ref, fcb_ref, out_ref,
           sums_sc, cnt_sc):
    i = pl.program_id(0)

    @pl.when(i == 0)
    def _():
      sums_sc[...] = jnp.zeros_like(sums_sc)
      cnt_sc[...] = jnp.zeros_like(cnt_sc)

    deg = degn_ref[:, 0:1] + 1.0
    dinv = lax.rsqrt(jnp.maximum(deg, 1.0))

    h = jnp.zeros((bn, h_dim), jnp.float32)
    for t in range(t_steps):
      agg = dinv * (p_ref[t] + y_ref[t])
      zt = jnp.maximum(dot_t(agg, wg_ref[...]) + bg_ref[...], 0.0)
      gi = dot_t(zt, wih_ref[...]) + bih_ref[...]
      gh = dot_t(h, whh_ref[...]) + bhh_ref[...]
      r = jax.nn.sigmoid(gi[:, :h_dim] + gh[:, :h_dim])
      z = jax.nn.sigmoid(gi[:, h_dim:2 * h_dim] + gh[:, h_dim:2 * h_dim])
      nn_ = jnp.tanh(gi[:, 2 * h_dim:] + r * gh[:, 2 * h_dim:])
      h = (1.0 - z) * nn_ + z * h

    out = dot_t(h, fcw_ref[...]) + fcb_ref[...]

    g_iota = lax.broadcasted_iota(jnp.int32, (bn, g_segs), 1)
    oneh = (batch_ref[...] == g_iota).astype(jnp.float32)
    part = lax.dot_general(oneh, out, (((0,), (0,)), ((), ())),
                           preferred_element_type=jnp.float32)
    pcnt = lax.dot_general(oneh, jnp.ones((bn, c_dim), jnp.float32),
                           (((0,), (0,)), ((), ())),
                           preferred_element_type=jnp.float32)
    sums_sc[...] += part
    cnt_sc[...] += pcnt
    out_ref[...] = sums_sc[...] / jnp.maximum(cnt_sc[...], 1.0)

  full = lambda shape: pl.BlockSpec(shape, lambda i: tuple(0 for _ in shape))
  return pl.pallas_call(
      body,
      grid=(nblk,),
      in_specs=[
          pl.BlockSpec((t_steps, bn, d), lambda i: (0, i, 0)),
          pl.BlockSpec((t_steps, bn, d), lambda i: (0, i, 0)),
          pl.BlockSpec((bn, d), lambda i: (i, 0)),
          pl.BlockSpec((bn, 1), lambda i: (i, 0)),
          full((h_dim, d)),
          full((1, h_dim)),
          full((3 * h_dim, h_dim)),
          full((3 * h_dim, h_dim)),
          full((1, 3 * h_dim)),
          full((1, 3 * h_dim)),
          full((c_dim, h_dim)),
          full((1, c_dim)),
      ],
      out_specs=pl.BlockSpec((g_segs, c_dim), lambda i: (0, 0)),
      out_shape=jax.ShapeDtypeStruct((g_segs, c_dim), jnp.float32),
      scratch_shapes=[
          pltpu.VMEM((g_segs, c_dim), jnp.float32),
          pltpu.VMEM((g_segs, c_dim), jnp.float32),
      ],
  )(y, p, degn, batch_pad, W_gcn, b_gcn, W_ih, W_hh, b_ih, b_hh, fc_W, fc_b)


def kernel(x_seq, edge_index, batch, W_gcn, b_gcn, W_ih, W_hh, b_ih, b_hh,
           fc_W, fc_b):
  t_steps, n, d = x_seq.shape
  e = edge_index.shape[1]
  g_segs = 64
  n_pad = 10240

  rpt = e // (NS * EB)
  src_rows = edge_index[0].reshape(NS, rpt, EB)
  dst_rows = edge_index[1].reshape(NS, rpt, EB)

  dummy = jnp.zeros((8, d), jnp.float32)
  degn = _sc_aggregate(dummy, dst_rows, dst_rows, 1, n_pad, d,
                       do_gather=False)[0]

  x_pad = jnp.pad(x_seq, ((0, 0), (0, n_pad - n), (0, 0)))
  y = _tc_scale(x_pad, degn)

  y_flat = y.reshape(t_steps * n_pad, d)
  p = _sc_aggregate(y_flat, src_rows, dst_rows, t_steps, n_pad, d)

  batch_pad = jnp.pad(batch, (0, n_pad - n),
                      constant_values=g_segs).reshape(n_pad, 1)
  b_gcn2 = b_gcn.reshape(1, -1)
  b_ih2 = b_ih.reshape(1, -1)
  b_hh2 = b_hh.reshape(1, -1)
  fc_b2 = fc_b.reshape(1, -1)

  return _tc_fused(y, p, degn, batch_pad, W_gcn, b_gcn2, W_ih, W_hh,
                   b_ih2, b_hh2, fc_W, fc_b2, g_segs)
